# baseline (speedup 1.0000x reference)
import jax
import jax.numpy as jnp
from jax import lax
from jax.experimental import pallas as pl
from jax.experimental.pallas import tpu as pltpu
from jax.experimental.pallas import tpu_sc as plsc

N_NODES = 1000000
NBX = NBY = 1024
PAIR_COLS = NBY // 2
MAP_WORDS = NBX * PAIR_COLS

NUM_CORES = 2
NUM_SUBCORES = 16
NW = NUM_CORES * NUM_SUBCORES
LANES = 16

B = 1664
NBLK = 19
C = B * NBLK
NP = NW * C

HI_MASK = -65536


def _lo(w):
    return lax.bitcast_convert_type(w << 16, jnp.float32)


def _hi(w):
    return lax.bitcast_convert_type(w & HI_MASK, jnp.float32)


def _body(xs, ys, sxs, sys_, pws, map_hbm, out_hbm,
          map_sp, xb, yb, sxb, syb, pwb, idxb, ub, oxb, valb, outb, sem):
    cid = lax.axis_index("c")
    sid = lax.axis_index("s")
    wid = sid * NUM_CORES + cid

    @pl.when(sid == 0)
    def _():
        pltpu.sync_copy(map_hbm, map_sp)

    plsc.subcore_barrier()

    def block(blk, _):
        base = wid * C + blk * B
        pltpu.sync_copy(xs.at[pl.ds(base, B)], xb)
        pltpu.sync_copy(ys.at[pl.ds(base, B)], yb)
        pltpu.sync_copy(sxs.at[pl.ds(base, B)], sxb)
        pltpu.sync_copy(sys_.at[pl.ds(base, B)], syb)
        pltpu.sync_copy(pws.at[pl.ds(base, B)], pwb)

        def gen(c, _):
            o = c * LANES
            x = xb[pl.ds(o, LANES)]
            y = yb[pl.ds(o, LANES)]
            sx = sxb[pl.ds(o, LANES)]
            sy = syb[pl.ds(o, LANES)]
            x2 = x + sx
            y2 = y + sy
            bxl = x.astype(jnp.int32)
            byl = y.astype(jnp.int32)
            bxf = bxl.astype(jnp.float32)
            byf = byl.astype(jnp.float32)
            ox = [jnp.maximum(
                jnp.minimum(x2, bxf + (d + 1.0)) - jnp.maximum(x, bxf + float(d)),
                0.0) for d in range(3)]
            oy = [jnp.maximum(
                jnp.minimum(y2, byf + (d + 1.0)) - jnp.maximum(y, byf + float(d)),
                0.0) for d in range(3)]
            byh = (y * 0.5).astype(jnp.int32)
            pb = bxl * PAIR_COLS + byh
            odd = (byl - 2 * byh).astype(jnp.float32)
            even = 1.0 - odd
            u = [even * oy[0],
                 even * oy[1] + odd * oy[0],
                 even * oy[2] + odd * oy[1],
                 odd * oy[2]]
            for dx in range(3):
                for j in range(2):
                    k = dx * 2 + j
                    idxb[pl.ds(k * B + o, LANES)] = pb + (dx * PAIR_COLS + j)
            for q in range(4):
                ub[pl.ds(q * B + o, LANES)] = u[q]
            for r in range(3):
                oxb[pl.ds(r * B + o, LANES)] = ox[r]
            return 0

        lax.fori_loop(0, B // LANES, gen, 0)

        pltpu.async_copy(map_sp.at[idxb], valb, sem).wait()

        def acc(c, _):
            o = c * LANES
            u0 = ub[pl.ds(0 * B + o, LANES)]
            u1 = ub[pl.ds(1 * B + o, LANES)]
            u2 = ub[pl.ds(2 * B + o, LANES)]
            u3 = ub[pl.ds(3 * B + o, LANES)]
            s = jnp.zeros((LANES,), jnp.float32)
            for dx in range(3):
                w0 = valb[pl.ds((dx * 2) * B + o, LANES)]
                w1 = valb[pl.ds((dx * 2 + 1) * B + o, LANES)]
                inner = (_lo(w0) * u0 + _hi(w0) * u1
                         + _lo(w1) * u2 + _hi(w1) * u3)
                s = s + oxb[pl.ds(dx * B + o, LANES)] * inner
            sx = sxb[pl.ds(o, LANES)]
            sy = syb[pl.ds(o, LANES)]
            pw = pwb[pl.ds(o, LANES)]
            outb[pl.ds(o, LANES)] = s * (10.0 * pw) / (sx * sy)
            return 0

        lax.fori_loop(0, B // LANES, acc, 0)

        pltpu.sync_copy(outb, out_hbm.at[pl.ds(base, B)])
        return 0

    lax.fori_loop(0, NBLK, block, 0)


@jax.jit
def _run(xs, ys, sxs, sys_, pws, map_words):
    mesh = plsc.VectorSubcoreMesh(core_axis_name="c", subcore_axis_name="s")
    return pl.kernel(
        _body,
        out_type=jax.ShapeDtypeStruct((NP,), jnp.float32),
        mesh=mesh,
        scratch_types=[
            pltpu.VMEM_SHARED((MAP_WORDS,), jnp.int32),
            pltpu.VMEM((B,), jnp.float32),
            pltpu.VMEM((B,), jnp.float32),
            pltpu.VMEM((B,), jnp.float32),
            pltpu.VMEM((B,), jnp.float32),
            pltpu.VMEM((B,), jnp.float32),
            pltpu.VMEM((6 * B,), jnp.int32),
            pltpu.VMEM((4 * B,), jnp.float32),
            pltpu.VMEM((3 * B,), jnp.float32),
            pltpu.VMEM((6 * B,), jnp.int32),
            pltpu.VMEM((B,), jnp.float32),
            pltpu.SemaphoreType.DMA,
        ],
    )(xs, ys, sxs, sys_, pws, map_words)


def kernel(pos, node_size_x, node_size_y, utilization_map, pin_weights):
    n = N_NODES
    pad = NP - n
    x = jnp.concatenate([pos[:n], jnp.zeros((pad,), jnp.float32)])
    y = jnp.concatenate([pos[n:2 * n], jnp.zeros((pad,), jnp.float32)])
    sx = jnp.concatenate([node_size_x[:n], jnp.ones((pad,), jnp.float32)])
    sy = jnp.concatenate([node_size_y[:n], jnp.ones((pad,), jnp.float32)])
    pw = jnp.concatenate([pin_weights[:n], jnp.zeros((pad,), jnp.float32)])
    mb = utilization_map.astype(jnp.bfloat16).reshape(-1)
    bits = lax.bitcast_convert_type(mb, jnp.uint16).astype(jnp.uint32)
    bits = bits.reshape(-1, 2)
    words = lax.bitcast_convert_type(bits[:, 0] | (bits[:, 1] << 16),
                                     jnp.int32)
    out = _run(x, y, sx, sy, pw, words)
    return out[:n]


# speedup vs baseline: 1.0035x; 1.0035x over previous
import jax
import jax.numpy as jnp
from jax import lax
from jax.experimental import pallas as pl
from jax.experimental.pallas import tpu as pltpu
from jax.experimental.pallas import tpu_sc as plsc

N_NODES = 1000000
NBX = NBY = 1024
PAIR_COLS = NBY // 2
MAP_WORDS = NBX * PAIR_COLS

NUM_CORES = 2
NUM_SUBCORES = 16
NW = NUM_CORES * NUM_SUBCORES
LANES = 16

B = 1664
NBLK = 19
C = B * NBLK
NP = NW * C

HI_MASK = -65536


def _lo(w):
    return w


def _hi(w):
    return w


def _body(xs, ys, sxs, sys_, pws, map_hbm, out_hbm,
          map_sp, xb, yb, sxb, syb, pwb, idxb, ub, oxb, valb, outb, sem):
    cid = lax.axis_index("c")
    sid = lax.axis_index("s")
    wid = sid * NUM_CORES + cid

    @pl.when(sid == 0)
    def _():
        pltpu.sync_copy(map_hbm, map_sp)

    plsc.subcore_barrier()

    def block(blk, _):
        base = wid * C + blk * B
        pltpu.sync_copy(xs.at[pl.ds(base, B)], xb)
        pltpu.sync_copy(ys.at[pl.ds(base, B)], yb)
        pltpu.sync_copy(sxs.at[pl.ds(base, B)], sxb)
        pltpu.sync_copy(sys_.at[pl.ds(base, B)], syb)
        pltpu.sync_copy(pws.at[pl.ds(base, B)], pwb)

        def gen(c, _):
            o = c * LANES
            x = xb[pl.ds(o, LANES)]
            y = yb[pl.ds(o, LANES)]
            sx = sxb[pl.ds(o, LANES)]
            sy = syb[pl.ds(o, LANES)]
            x2 = x + sx
            y2 = y + sy
            bxl = x.astype(jnp.int32)
            byl = y.astype(jnp.int32)
            bxf = bxl.astype(jnp.float32)
            byf = byl.astype(jnp.float32)
            ox = [jnp.maximum(
                jnp.minimum(x2, bxf + (d + 1.0)) - jnp.maximum(x, bxf + float(d)),
                0.0) for d in range(3)]
            oy = [jnp.maximum(
                jnp.minimum(y2, byf + (d + 1.0)) - jnp.maximum(y, byf + float(d)),
                0.0) for d in range(3)]
            byh = (y * 0.5).astype(jnp.int32)
            pb = bxl * PAIR_COLS + byh
            odd = (byl - 2 * byh).astype(jnp.float32)
            even = 1.0 - odd
            u = [even * oy[0],
                 even * oy[1] + odd * oy[0],
                 even * oy[2] + odd * oy[1],
                 odd * oy[2]]
            for dx in range(3):
                for j in range(2):
                    k = dx * 2 + j
                    idxb[pl.ds(k * B + o, LANES)] = pb + (dx * PAIR_COLS + j)
            for q in range(4):
                ub[pl.ds(q * B + o, LANES)] = u[q]
            for r in range(3):
                oxb[pl.ds(r * B + o, LANES)] = ox[r]
            return 0

        lax.fori_loop(0, B // LANES, gen, 0)

        pltpu.async_copy(map_sp.at[idxb], valb, sem).wait()

        def acc(c, _):
            o = c * LANES
            u0 = ub[pl.ds(0 * B + o, LANES)]
            u1 = ub[pl.ds(1 * B + o, LANES)]
            u2 = ub[pl.ds(2 * B + o, LANES)]
            u3 = ub[pl.ds(3 * B + o, LANES)]
            s = jnp.zeros((LANES,), jnp.float32)
            for dx in range(3):
                w0 = valb[pl.ds((dx * 2) * B + o, LANES)]
                w1 = valb[pl.ds((dx * 2 + 1) * B + o, LANES)]
                inner = (_lo(w0) * u0 + _hi(w0) * u1
                         + _lo(w1) * u2 + _hi(w1) * u3)
                s = s + oxb[pl.ds(dx * B + o, LANES)] * inner
            sx = sxb[pl.ds(o, LANES)]
            sy = syb[pl.ds(o, LANES)]
            pw = pwb[pl.ds(o, LANES)]
            outb[pl.ds(o, LANES)] = s * (10.0 * pw) / (sx * sy)
            return 0

        lax.fori_loop(0, B // LANES, acc, 0)

        pltpu.sync_copy(outb, out_hbm.at[pl.ds(base, B)])
        return 0

    lax.fori_loop(0, NBLK, block, 0)


@jax.jit
def _run(xs, ys, sxs, sys_, pws, map_words):
    mesh = plsc.VectorSubcoreMesh(core_axis_name="c", subcore_axis_name="s")
    return pl.kernel(
        _body,
        out_type=jax.ShapeDtypeStruct((NP,), jnp.float32),
        mesh=mesh,
        scratch_types=[
            pltpu.VMEM_SHARED((MAP_WORDS,), jnp.float32),
            pltpu.VMEM((B,), jnp.float32),
            pltpu.VMEM((B,), jnp.float32),
            pltpu.VMEM((B,), jnp.float32),
            pltpu.VMEM((B,), jnp.float32),
            pltpu.VMEM((B,), jnp.float32),
            pltpu.VMEM((6 * B,), jnp.int32),
            pltpu.VMEM((4 * B,), jnp.float32),
            pltpu.VMEM((3 * B,), jnp.float32),
            pltpu.VMEM((6 * B,), jnp.float32),
            pltpu.VMEM((B,), jnp.float32),
            pltpu.SemaphoreType.DMA,
        ],
    )(xs, ys, sxs, sys_, pws, map_words)


def kernel(pos, node_size_x, node_size_y, utilization_map, pin_weights):
    n = N_NODES
    pad = NP - n
    x = jnp.concatenate([pos[:n], jnp.zeros((pad,), jnp.float32)])
    y = jnp.concatenate([pos[n:2 * n], jnp.zeros((pad,), jnp.float32)])
    sx = jnp.concatenate([node_size_x[:n], jnp.ones((pad,), jnp.float32)])
    sy = jnp.concatenate([node_size_y[:n], jnp.ones((pad,), jnp.float32)])
    pw = jnp.concatenate([pin_weights[:n], jnp.zeros((pad,), jnp.float32)])
    mb = utilization_map.astype(jnp.bfloat16).reshape(-1)
    bits = lax.bitcast_convert_type(mb, jnp.uint16).astype(jnp.uint32)
    bits = bits.reshape(-1, 2)
    words = lax.bitcast_convert_type(bits[:, 0] | (bits[:, 1] << 16),
                                     jnp.float32)
    out = _run(x, y, sx, sy, pw, words)
    return out[:n]


# speedup vs baseline: 2.2352x; 2.2275x over previous
import functools

import jax
import jax.numpy as jnp
from jax import lax
from jax.experimental import pallas as pl
from jax.experimental.pallas import tpu as pltpu
from jax.experimental.pallas import tpu_sc as plsc

N_NODES = 1000000
NBX = NBY = 1024
MAP_WORDS = NBX * NBY

NUM_CORES = 2
NUM_SUBCORES = 16
NW = NUM_CORES * NUM_SUBCORES
LANES = 16

B = 1664
NBLK = 19
C = B * NBLK
NP = NW * C
NR = 9 * B // 128


def _body(xs, ys, sxs, sys_, pws, map_hbm, out_hbm,
          map_sp, xb, yb, sxb, syb, pwb, idxb, wb, valb, outb, sem):
    cid = lax.axis_index("c")
    sid = lax.axis_index("s")
    wid = sid * NUM_CORES + cid

    @pl.when(sid == 0)
    def _():
        pltpu.sync_copy(map_hbm, map_sp)

    plsc.subcore_barrier()

    def block(blk, _):
        base = wid * C + blk * B
        pltpu.sync_copy(xs.at[pl.ds(base, B)], xb)
        pltpu.sync_copy(ys.at[pl.ds(base, B)], yb)
        pltpu.sync_copy(sxs.at[pl.ds(base, B)], sxb)
        pltpu.sync_copy(sys_.at[pl.ds(base, B)], syb)
        pltpu.sync_copy(pws.at[pl.ds(base, B)], pwb)

        def gen(c, _):
            x = xb[pl.ds(c * LANES, LANES)]
            y = yb[pl.ds(c * LANES, LANES)]
            sx = sxb[pl.ds(c * LANES, LANES)]
            sy = syb[pl.ds(c * LANES, LANES)]
            x2 = x + sx
            y2 = y + sy
            bxl = x.astype(jnp.int32)
            byl = y.astype(jnp.int32)
            bxf = bxl.astype(jnp.float32)
            byf = byl.astype(jnp.float32)
            ox = [jnp.maximum(
                jnp.minimum(x2, bxf + (d + 1.0)) - jnp.maximum(x, bxf + float(d)),
                0.0) for d in range(3)]
            oy = [jnp.maximum(
                jnp.minimum(y2, byf + (d + 1.0)) - jnp.maximum(y, byf + float(d)),
                0.0) for d in range(3)]
            fb = bxl * NBX + byl
            o = c * LANES
            for dx in range(3):
                for dy in range(3):
                    k = dx * 3 + dy
                    idxb[pl.ds(k * B + o, LANES)] = fb + (dx * NBX + dy)
                    wb[pl.ds(k * B + o, LANES)] = ox[dx] * oy[dy]
            return 0

        lax.fori_loop(0, B // LANES, gen, 0)

        pltpu.async_copy(map_sp.at[idxb], valb, sem).wait()

        def acc(c, _):
            o = c * LANES
            s = jnp.zeros((LANES,), jnp.float32)
            for k in range(9):
                s = s + valb[pl.ds(k * B + o, LANES)] * wb[pl.ds(k * B + o, LANES)]
            sx = sxb[pl.ds(c * LANES, LANES)]
            sy = syb[pl.ds(c * LANES, LANES)]
            pw = pwb[pl.ds(c * LANES, LANES)]
            outb[pl.ds(c * LANES, LANES)] = s * (10.0 * pw) / (sx * sy)
            return 0

        lax.fori_loop(0, B // LANES, acc, 0)

        pltpu.sync_copy(outb, out_hbm.at[pl.ds(base, B)])
        return 0

    lax.fori_loop(0, NBLK, block, 0)


@jax.jit
def _run(xs, ys, sxs, sys_, pws, map_flat):
    mesh = plsc.VectorSubcoreMesh(core_axis_name="c", subcore_axis_name="s")
    return pl.kernel(
        _body,
        out_type=jax.ShapeDtypeStruct((NP,), jnp.float32),
        mesh=mesh,
        scratch_types=[
            pltpu.VMEM_SHARED((MAP_WORDS,), jnp.float32),
            pltpu.VMEM((B,), jnp.float32),
            pltpu.VMEM((B,), jnp.float32),
            pltpu.VMEM((B,), jnp.float32),
            pltpu.VMEM((B,), jnp.float32),
            pltpu.VMEM((B,), jnp.float32),
            pltpu.VMEM((9 * B,), jnp.int32),
            pltpu.VMEM((9 * B,), jnp.float32),
            pltpu.VMEM((9 * B,), jnp.float32),
            pltpu.VMEM((B,), jnp.float32),
            pltpu.SemaphoreType.DMA,
        ],
    )(xs, ys, sxs, sys_, pws, map_flat)


def kernel(pos, node_size_x, node_size_y, utilization_map, pin_weights):
    n = N_NODES
    pad = NP - n
    x = jnp.concatenate([pos[:n], jnp.zeros((pad,), jnp.float32)])
    y = jnp.concatenate([pos[n:2 * n], jnp.zeros((pad,), jnp.float32)])
    sx = jnp.concatenate([node_size_x[:n], jnp.ones((pad,), jnp.float32)])
    sy = jnp.concatenate([node_size_y[:n], jnp.ones((pad,), jnp.float32)])
    pw = jnp.concatenate([pin_weights[:n], jnp.zeros((pad,), jnp.float32)])
    out = _run(x, y, sx, sy, pw, utilization_map.reshape(-1))
    return out[:n]


# speedup vs baseline: 2.5170x; 1.1261x over previous
import jax
import jax.numpy as jnp
from jax import lax
from jax.experimental import pallas as pl
from jax.experimental.pallas import tpu as pltpu
from jax.experimental.pallas import tpu_sc as plsc

N_NODES = 1000000
NBX = NBY = 1024
PAIR_ROWS = NBX // 2
MAP_WORDS = PAIR_ROWS * NBY

NUM_CORES = 2
NUM_SUBCORES = 16
NW = NUM_CORES * NUM_SUBCORES
LANES = 16

B = 1664
NBLK = 19
C = B * NBLK
NP = NW * C

HI_MASK = -65536


def _lo(w):
    return lax.bitcast_convert_type(w << 16, jnp.float32)


def _hi(w):
    return lax.bitcast_convert_type(w & HI_MASK, jnp.float32)


def _body(xs, ys, sxs, sys_, pws, map_hbm, out_hbm,
          map_sp, xb, yb, sxb, syb, pwb, idxb, ub, oyb, valb, outb, sem):
    cid = lax.axis_index("c")
    sid = lax.axis_index("s")
    wid = sid * NUM_CORES + cid

    @pl.when(sid == 0)
    def _():
        pltpu.sync_copy(map_hbm, map_sp)

    plsc.subcore_barrier()

    def block(blk, _):
        base = wid * C + blk * B
        pltpu.sync_copy(xs.at[pl.ds(base, B)], xb)
        pltpu.sync_copy(ys.at[pl.ds(base, B)], yb)
        pltpu.sync_copy(sxs.at[pl.ds(base, B)], sxb)
        pltpu.sync_copy(sys_.at[pl.ds(base, B)], syb)
        pltpu.sync_copy(pws.at[pl.ds(base, B)], pwb)

        def gen(c, _):
            o = c * LANES
            x = xb[pl.ds(o, LANES)]
            y = yb[pl.ds(o, LANES)]
            sx = sxb[pl.ds(o, LANES)]
            sy = syb[pl.ds(o, LANES)]
            x2 = x + sx
            y2 = y + sy
            bxl = x.astype(jnp.int32)
            byl = y.astype(jnp.int32)
            bxf = bxl.astype(jnp.float32)
            byf = byl.astype(jnp.float32)
            ox = [jnp.maximum(
                jnp.minimum(x2, bxf + (d + 1.0)) - jnp.maximum(x, bxf + float(d)),
                0.0) for d in range(3)]
            oy = [jnp.maximum(
                jnp.minimum(y2, byf + (d + 1.0)) - jnp.maximum(y, byf + float(d)),
                0.0) for d in range(3)]
            bxh = (x * 0.5).astype(jnp.int32)
            pb = bxh * NBY + byl
            odd = (bxl - 2 * bxh).astype(jnp.float32)
            even = 1.0 - odd
            u = [even * ox[0],
                 even * ox[1] + odd * ox[0],
                 even * ox[2] + odd * ox[1],
                 odd * ox[2]]
            for dy in range(3):
                for j in range(2):
                    k = dy * 2 + j
                    idxb[pl.ds(k * B + o, LANES)] = pb + (j * NBY + dy)
            for q in range(4):
                ub[pl.ds(q * B + o, LANES)] = u[q]
            for r in range(3):
                oyb[pl.ds(r * B + o, LANES)] = oy[r]
            return 0

        lax.fori_loop(0, B // LANES, gen, 0)

        pltpu.async_copy(map_sp.at[idxb], valb, sem).wait()

        def acc(c, _):
            o = c * LANES
            u0 = ub[pl.ds(0 * B + o, LANES)]
            u1 = ub[pl.ds(1 * B + o, LANES)]
            u2 = ub[pl.ds(2 * B + o, LANES)]
            u3 = ub[pl.ds(3 * B + o, LANES)]
            s = jnp.zeros((LANES,), jnp.float32)
            for dy in range(3):
                w0 = valb[pl.ds((dy * 2) * B + o, LANES)]
                w1 = valb[pl.ds((dy * 2 + 1) * B + o, LANES)]
                inner = (_lo(w0) * u0 + _hi(w0) * u1
                         + _lo(w1) * u2 + _hi(w1) * u3)
                s = s + oyb[pl.ds(dy * B + o, LANES)] * inner
            sx = sxb[pl.ds(o, LANES)]
            sy = syb[pl.ds(o, LANES)]
            pw = pwb[pl.ds(o, LANES)]
            outb[pl.ds(o, LANES)] = s * (10.0 * pw) / (sx * sy)
            return 0

        lax.fori_loop(0, B // LANES, acc, 0)

        pltpu.sync_copy(outb, out_hbm.at[pl.ds(base, B)])
        return 0

    lax.fori_loop(0, NBLK, block, 0)


@jax.jit
def _run(xs, ys, sxs, sys_, pws, map_words):
    mesh = plsc.VectorSubcoreMesh(core_axis_name="c", subcore_axis_name="s")
    return pl.kernel(
        _body,
        out_type=jax.ShapeDtypeStruct((NP,), jnp.float32),
        mesh=mesh,
        scratch_types=[
            pltpu.VMEM_SHARED((MAP_WORDS,), jnp.int32),
            pltpu.VMEM((B,), jnp.float32),
            pltpu.VMEM((B,), jnp.float32),
            pltpu.VMEM((B,), jnp.float32),
            pltpu.VMEM((B,), jnp.float32),
            pltpu.VMEM((B,), jnp.float32),
            pltpu.VMEM((6 * B,), jnp.int32),
            pltpu.VMEM((4 * B,), jnp.float32),
            pltpu.VMEM((3 * B,), jnp.float32),
            pltpu.VMEM((6 * B,), jnp.int32),
            pltpu.VMEM((B,), jnp.float32),
            pltpu.SemaphoreType.DMA,
        ],
    )(xs, ys, sxs, sys_, pws, map_words)


def kernel(pos, node_size_x, node_size_y, utilization_map, pin_weights):
    n = N_NODES
    pad = NP - n
    x = jnp.concatenate([pos[:n], jnp.zeros((pad,), jnp.float32)])
    y = jnp.concatenate([pos[n:2 * n], jnp.zeros((pad,), jnp.float32)])
    sx = jnp.concatenate([node_size_x[:n], jnp.ones((pad,), jnp.float32)])
    sy = jnp.concatenate([node_size_y[:n], jnp.ones((pad,), jnp.float32)])
    pw = jnp.concatenate([pin_weights[:n], jnp.zeros((pad,), jnp.float32)])
    b = lax.bitcast_convert_type(utilization_map, jnp.int32)
    r = b + 32767 + ((b >> 16) & 1)
    rlo = r[0::2, :]
    rhi = r[1::2, :]
    words = ((rlo >> 16) | (rhi & HI_MASK)).reshape(-1)
    out = _run(x, y, sx, sy, pw, words)
    return out[:n]
